# baseline (speedup 1.0000x reference)
import functools

import jax
import jax.numpy as jnp
from jax import lax
from jax.experimental import pallas as pl
from jax.experimental.pallas import tpu as pltpu
from jax.experimental.pallas import tpu_sc as plsc

_PRE = 2048
_POST = 512
_THRESH = 0.7
_NP = 20480
_B = 4
_PW = _NP // 16
_VW = _PW // 16


def _lane_sum(v):
    return jnp.sum(v)


def _pick(vec, lane, iot):
    return jnp.sum(jnp.where(iot == lane, vec, jnp.zeros_like(vec)))


def _sc_select_kernel(scores_hbm, out_hbm,
                      fbuf, kbuf, histv, gridv, localimg, accbuf, tmp128,
                      histgrid_sp, imggrid_sp):
    cid = lax.axis_index("c")
    sid = lax.axis_index("s")
    iot = lax.iota(jnp.int32, 16)
    zeros16 = jnp.zeros((16,), jnp.int32)

    for bi in [cid]:
        base = bi * _NP + sid * _PW

        pltpu.sync_copy(scores_hbm.at[pl.ds(base, _PW)], fbuf)

        def kbody(i, _):
            v = fbuf[pl.ds(i * 16, 16)]
            b32 = lax.bitcast_convert_type(v, jnp.int32)
            flip = jnp.right_shift(b32, 31) & 0x7FFFFFFF
            kbuf[pl.ds(i * 16, 16)] = b32 ^ flip
            return 0

        lax.fori_loop(0, _VW, kbody, 0)

        prefix = jnp.int32(0)
        remaining = jnp.int32(_PRE)
        for p in range(8):
            shift = 28 - 4 * p
            himask = jnp.int32(-(1 << (shift + 4))) if p > 0 else jnp.int32(0)

            def hbody(i, accs, _himask=himask, _shift=shift, _prefix=prefix,
                      _p=p):
                dk = kbuf[pl.ds(i * 16, 16)] ^ jnp.int32(-2147483648)
                if _p == 0:
                    match = jnp.ones((16,), jnp.bool_)
                else:
                    match = ((dk ^ _prefix) & _himask) == 0
                digit = jnp.right_shift(dk, _shift) & 15
                return tuple(
                    accs[d] + jnp.where((digit == d) & match,
                                        jnp.ones((16,), jnp.int32), zeros16)
                    for d in range(16))

            accs = lax.fori_loop(0, _VW, hbody, tuple(zeros16 for _ in range(16)))
            h = zeros16
            for d in range(16):
                h = h + jnp.where(iot == d,
                                  jnp.full((16,), _lane_sum(accs[d]), jnp.int32),
                                  zeros16)
            histv[...] = h
            pltpu.sync_copy(histv, histgrid_sp.at[sid])
            plsc.subcore_barrier()
            pltpu.sync_copy(histgrid_sp, gridv)
            g = zeros16
            for w in range(16):
                g = g + gridv[w]
            plsc.subcore_barrier()

            c = plsc.cumsum(g)
            total = jnp.max(c)
            cum_t = total - c + g
            maskv = cum_t >= remaining
            dstar = jnp.max(jnp.where(maskv, iot, jnp.full((16,), -1, jnp.int32)))
            c_at = _pick(c, dstar, iot)
            remaining = remaining - (total - c_at)
            prefix = prefix | (dstar << shift)

        tau = prefix ^ jnp.int32(-2147483648)
        tfin = remaining

        def cbody(i, carry):
            aG, aT = carry
            k = kbuf[pl.ds(i * 16, 16)]
            aG = aG + jnp.where(k > tau, jnp.ones((16,), jnp.int32), zeros16)
            aT = aT + jnp.where(k == tau, jnp.ones((16,), jnp.int32), zeros16)
            return aG, aT

        aG, aT = lax.fori_loop(0, _VW, cbody, (zeros16, zeros16))
        cntG = _lane_sum(aG)
        cntT = _lane_sum(aT)
        histv[...] = (jnp.where(iot == 0, jnp.full((16,), cntG, jnp.int32), zeros16)
                      + jnp.where(iot == 1, jnp.full((16,), cntT, jnp.int32), zeros16))
        pltpu.sync_copy(histv, histgrid_sp.at[sid])
        plsc.subcore_barrier()
        pltpu.sync_copy(histgrid_sp, gridv)
        cntG_v = plsc.load_gather(gridv, [iot, zeros16])
        cntT_v = plsc.load_gather(gridv, [iot, jnp.ones((16,), jnp.int32)])
        prefT = plsc.cumsum(cntT_v) - cntT_v
        t_v = jnp.minimum(jnp.maximum(tfin - prefT, 0), cntT_v)
        m_v = cntG_v + t_v
        prefM = plsc.cumsum(m_v) - m_v
        t_w = _pick(t_v, sid, iot)
        outbase = _pick(prefM, sid, iot)
        plsc.subcore_barrier()

        def zbody(i, _):
            localimg[pl.ds(i * 16, 16)] = zeros16
            return 0

        lax.fori_loop(0, _PRE // 16, zbody, 0)

        def dbody(i, carry):
            rM, rT = carry
            k = kbuf[pl.ds(i * 16, 16)]
            isG = k > tau
            ti = jnp.where(k == tau, jnp.ones((16,), jnp.int32), zeros16)
            exT = plsc.cumsum(ti) - ti
            takeT = (k == tau) & ((rT + exT) < t_w)
            member = isG | takeT
            mi = jnp.where(member, jnp.ones((16,), jnp.int32), zeros16)
            exM = plsc.cumsum(mi) - mi
            pos = jnp.where(member, outbase + rM + exM, zeros16)
            orig = sid * _PW + i * 16 + iot
            plsc.store_scatter(localimg, [pos], orig, mask=member)
            return rM + _lane_sum(mi), rT + _lane_sum(ti)

        lax.fori_loop(0, _VW, dbody, (jnp.int32(0), jnp.int32(0)))
        pltpu.sync_copy(localimg, imggrid_sp.at[sid])
        plsc.subcore_barrier()

        def z2body(i, _):
            accbuf[pl.ds(i * 16, 16)] = zeros16
            return 0

        lax.fori_loop(0, 128 // 16, z2body, 0)
        for v in range(16):
            pltpu.sync_copy(imggrid_sp.at[v, pl.ds(sid * 128, 128)], tmp128)
            for j in range(128 // 16):
                accbuf[pl.ds(j * 16, 16)] = (accbuf[pl.ds(j * 16, 16)]
                                             + tmp128[pl.ds(j * 16, 16)])
        pltpu.sync_copy(accbuf, out_hbm.at[pl.ds(bi * _PRE + sid * 128, 128)])
        plsc.subcore_barrier()


def _sc_select(scores_flat):
    mesh = plsc.VectorSubcoreMesh(core_axis_name="c", subcore_axis_name="s",
                                  num_cores=2, num_subcores=16)
    f = pl.kernel(
        _sc_select_kernel,
        mesh=mesh,
        compiler_params=pltpu.CompilerParams(needs_layout_passes=False),
        out_type=jax.ShapeDtypeStruct((2 * _PRE,), jnp.int32),
        scratch_types=[
            pltpu.VMEM((_PW,), jnp.float32),
            pltpu.VMEM((_PW,), jnp.int32),
            pltpu.VMEM((16,), jnp.int32),
            pltpu.VMEM((16, 16), jnp.int32),
            pltpu.VMEM((_PRE,), jnp.int32),
            pltpu.VMEM((128,), jnp.int32),
            pltpu.VMEM((128,), jnp.int32),
            pltpu.VMEM_SHARED((16, 16), jnp.int32),
            pltpu.VMEM_SHARED((16, _PRE), jnp.int32),
        ],
    )
    return f(scores_flat)


def _nms_kernel(tc_ref, tr_ref, rois_ref, rsc_ref, rlb_ref):
    gc = tc_ref[0]
    gr = tr_ref[0]
    dxc = jnp.abs(gc[:, 3:4])
    dyc = jnp.abs(gc[:, 4:5])
    x1c = gc[:, 0:1] - dxc * 0.5
    x2c = gc[:, 0:1] + dxc * 0.5
    y1c = gc[:, 1:2] - dyc * 0.5
    y2c = gc[:, 1:2] + dyc * 0.5
    ac = dxc * dyc
    dxr = jnp.abs(gr[3:4, :])
    dyr = jnp.abs(gr[4:5, :])
    x1r = gr[0:1, :] - dxr * 0.5
    x2r = gr[0:1, :] + dxr * 0.5
    y1r = gr[1:2, :] - dyr * 0.5
    y2r = gr[1:2, :] + dyr * 0.5
    ar = dxr * dyr
    ix = jnp.maximum(0.0, jnp.minimum(x2c, x2r) - jnp.maximum(x1c, x1r))
    iy = jnp.maximum(0.0, jnp.minimum(y2c, y2r) - jnp.maximum(y1c, y1r))
    inter = ix * iy
    iou = inter / jnp.maximum(ac + ar - inter, 1e-8)

    sc_col = gc[:, 7:8]
    sc_row = gr[7:8, :]
    ri = lax.broadcasted_iota(jnp.int32, (_PRE, _PRE), 0)
    ci = lax.broadcasted_iota(jnp.int32, (_PRE, _PRE), 1)
    dom = (sc_col > sc_row) | ((sc_col == sc_row) & (ri < ci))
    dom_f = jnp.where(dom, 1.0, 0.0)
    sup_m = jnp.where(dom & (iou > _THRESH), 1.0, 0.0)

    def cond(carry):
        return carry[1] == 1

    def body(carry):
        keep, _ = carry
        cnt = jnp.dot(keep, sup_m, preferred_element_type=jnp.float32)
        new = jnp.where(cnt > 0.0, 0.0, 1.0)
        changed = (jnp.sum(jnp.abs(new - keep)) > 0.0).astype(jnp.int32)
        return (new, changed)

    keep0 = jnp.ones((1, _PRE), jnp.float32)
    keep, _ = lax.while_loop(cond, body, (keep0, jnp.int32(1)))

    rank = jnp.dot(keep, dom_f, preferred_element_type=jnp.float32)
    slot = lax.broadcasted_iota(jnp.int32, (_POST, 1), 0).astype(jnp.float32)
    onehot = jnp.where((rank == slot) & (keep > 0.0), 1.0, 0.0)

    cols = [jnp.sum(onehot * gr[c:c + 1, :], axis=1, keepdims=True)
            for c in range(7)]
    rois_ref[0] = jnp.concatenate(cols, axis=1)
    rsc_ref[0] = jnp.sum(onehot * sc_row, axis=1, keepdims=True)
    lf = jnp.sum(onehot * gr[8:9, :], axis=1, keepdims=True)
    rlb_ref[0] = lf.astype(jnp.int32)


def _nms_from_member_table(table_col):
    B = table_col.shape[0]
    table_row = jnp.swapaxes(table_col, 1, 2)
    rois, rsc, rlb = pl.pallas_call(
        _nms_kernel,
        grid=(B,),
        in_specs=[
            pl.BlockSpec((1, _PRE, 9), lambda i: (i, 0, 0)),
            pl.BlockSpec((1, 9, _PRE), lambda i: (i, 0, 0)),
        ],
        out_specs=[
            pl.BlockSpec((1, _POST, 7), lambda i: (i, 0, 0)),
            pl.BlockSpec((1, _POST, 1), lambda i: (i, 0, 0)),
            pl.BlockSpec((1, _POST, 1), lambda i: (i, 0, 0)),
        ],
        out_shape=[
            jax.ShapeDtypeStruct((B, _POST, 7), jnp.float32),
            jax.ShapeDtypeStruct((B, _POST, 1), jnp.float32),
            jax.ShapeDtypeStruct((B, _POST, 1), jnp.int32),
        ],
    )(table_col, table_row)
    return rois, rsc[..., 0], rlb[..., 0]


def kernel(batch_box_preds, batch_cls_preds):
    B, N, _ = batch_box_preds.shape
    scores = jnp.max(batch_cls_preds, axis=-1)
    labels = jnp.argmax(batch_cls_preds, axis=-1)
    table = jnp.concatenate([
        batch_box_preds,
        scores[..., None],
        (labels + 1).astype(jnp.float32)[..., None],
    ], axis=-1)
    scores_pad = jnp.pad(scores, ((0, 0), (0, _NP - N)),
                         constant_values=-jnp.inf)
    outs = []
    for p in range(B // 2):
        midx = _sc_select(
            scores_pad[2 * p:2 * p + 2].reshape(-1)).reshape(2, _PRE)
        mt = jnp.take_along_axis(table[2 * p:2 * p + 2],
                                 midx[..., None], axis=1)
        outs.append(_nms_from_member_table(mt))
    return tuple(jnp.concatenate([o[i] for o in outs], axis=0)
                 for i in range(3))


# speedup vs baseline: 1.1565x; 1.1565x over previous
import functools

import jax
import jax.numpy as jnp
from jax import lax
from jax.experimental import pallas as pl
from jax.experimental.pallas import tpu as pltpu
from jax.experimental.pallas import tpu_sc as plsc

_PRE = 2048
_POST = 512
_THRESH = 0.7
_NP = 20480
_B = 4
_PW = _NP // 16
_VW = _PW // 16


def _lane_sum(v):
    return jnp.sum(v)


def _pick(vec, lane, iot):
    return jnp.sum(jnp.where(iot == lane, vec, jnp.zeros_like(vec)))


def _sc_select_kernel(scores_hbm, out_hbm,
                      fbuf, kbuf, histv, gridv, localimg, accbuf, tmp128,
                      histgrid_sp, imggrid_sp):
    cid = lax.axis_index("c")
    sid = lax.axis_index("s")
    iot = lax.iota(jnp.int32, 16)
    zeros16 = jnp.zeros((16,), jnp.int32)

    for b in range(2):
        bi = cid * 2 + b
        base = bi * _NP + sid * _PW

        pltpu.sync_copy(scores_hbm.at[pl.ds(base, _PW)], fbuf)

        def kbody(i, _):
            v = fbuf[pl.ds(i * 16, 16)]
            b32 = lax.bitcast_convert_type(v, jnp.int32)
            flip = jnp.right_shift(b32, 31) & 0x7FFFFFFF
            kbuf[pl.ds(i * 16, 16)] = b32 ^ flip
            return 0

        lax.fori_loop(0, _VW, kbody, 0)

        prefix = jnp.int32(0)
        remaining = jnp.int32(_PRE)
        for p in range(8):
            shift = 28 - 4 * p
            himask = jnp.int32(-(1 << (shift + 4))) if p > 0 else jnp.int32(0)

            def hbody(i, accs, _himask=himask, _shift=shift, _prefix=prefix,
                      _p=p):
                dk = kbuf[pl.ds(i * 16, 16)] ^ jnp.int32(-2147483648)
                if _p == 0:
                    match = jnp.ones((16,), jnp.bool_)
                else:
                    match = ((dk ^ _prefix) & _himask) == 0
                digit = jnp.right_shift(dk, _shift) & 15
                return tuple(
                    accs[d] + jnp.where((digit == d) & match,
                                        jnp.ones((16,), jnp.int32), zeros16)
                    for d in range(16))

            accs = lax.fori_loop(0, _VW, hbody, tuple(zeros16 for _ in range(16)))
            h = zeros16
            for d in range(16):
                h = h + jnp.where(iot == d,
                                  jnp.full((16,), _lane_sum(accs[d]), jnp.int32),
                                  zeros16)
            histv[...] = h
            pltpu.sync_copy(histv, histgrid_sp.at[sid])
            plsc.subcore_barrier()
            pltpu.sync_copy(histgrid_sp, gridv)
            g = zeros16
            for w in range(16):
                g = g + gridv[w]
            plsc.subcore_barrier()

            c = plsc.cumsum(g)
            total = jnp.max(c)
            cum_t = total - c + g
            maskv = cum_t >= remaining
            dstar = jnp.max(jnp.where(maskv, iot, jnp.full((16,), -1, jnp.int32)))
            c_at = _pick(c, dstar, iot)
            remaining = remaining - (total - c_at)
            prefix = prefix | (dstar << shift)

        tau = prefix ^ jnp.int32(-2147483648)
        tfin = remaining

        def cbody(i, carry):
            aG, aT = carry
            k = kbuf[pl.ds(i * 16, 16)]
            aG = aG + jnp.where(k > tau, jnp.ones((16,), jnp.int32), zeros16)
            aT = aT + jnp.where(k == tau, jnp.ones((16,), jnp.int32), zeros16)
            return aG, aT

        aG, aT = lax.fori_loop(0, _VW, cbody, (zeros16, zeros16))
        cntG = _lane_sum(aG)
        cntT = _lane_sum(aT)
        histv[...] = (jnp.where(iot == 0, jnp.full((16,), cntG, jnp.int32), zeros16)
                      + jnp.where(iot == 1, jnp.full((16,), cntT, jnp.int32), zeros16))
        pltpu.sync_copy(histv, histgrid_sp.at[sid])
        plsc.subcore_barrier()
        pltpu.sync_copy(histgrid_sp, gridv)
        cntG_v = plsc.load_gather(gridv, [iot, zeros16])
        cntT_v = plsc.load_gather(gridv, [iot, jnp.ones((16,), jnp.int32)])
        prefT = plsc.cumsum(cntT_v) - cntT_v
        t_v = jnp.minimum(jnp.maximum(tfin - prefT, 0), cntT_v)
        m_v = cntG_v + t_v
        prefM = plsc.cumsum(m_v) - m_v
        t_w = _pick(t_v, sid, iot)
        outbase = _pick(prefM, sid, iot)
        plsc.subcore_barrier()

        def zbody(i, _):
            localimg[pl.ds(i * 16, 16)] = zeros16
            return 0

        lax.fori_loop(0, _PRE // 16, zbody, 0)

        def dbody(i, carry):
            rM, rT = carry
            k = kbuf[pl.ds(i * 16, 16)]
            isG = k > tau
            ti = jnp.where(k == tau, jnp.ones((16,), jnp.int32), zeros16)
            exT = plsc.cumsum(ti) - ti
            takeT = (k == tau) & ((rT + exT) < t_w)
            member = isG | takeT
            mi = jnp.where(member, jnp.ones((16,), jnp.int32), zeros16)
            exM = plsc.cumsum(mi) - mi
            pos = jnp.where(member, outbase + rM + exM, zeros16)
            orig = sid * _PW + i * 16 + iot
            plsc.store_scatter(localimg, [pos], orig, mask=member)
            return rM + _lane_sum(mi), rT + _lane_sum(ti)

        lax.fori_loop(0, _VW, dbody, (jnp.int32(0), jnp.int32(0)))
        pltpu.sync_copy(localimg, imggrid_sp.at[sid])
        plsc.subcore_barrier()

        def z2body(i, _):
            accbuf[pl.ds(i * 16, 16)] = zeros16
            return 0

        lax.fori_loop(0, 128 // 16, z2body, 0)
        for v in range(16):
            pltpu.sync_copy(imggrid_sp.at[v, pl.ds(sid * 128, 128)], tmp128)
            for j in range(128 // 16):
                accbuf[pl.ds(j * 16, 16)] = (accbuf[pl.ds(j * 16, 16)]
                                             + tmp128[pl.ds(j * 16, 16)])
        pltpu.sync_copy(accbuf, out_hbm.at[pl.ds(bi * _PRE + sid * 128, 128)])
        plsc.subcore_barrier()


def _sc_select(scores_flat):
    mesh = plsc.VectorSubcoreMesh(core_axis_name="c", subcore_axis_name="s",
                                  num_cores=2, num_subcores=16)
    f = pl.kernel(
        _sc_select_kernel,
        mesh=mesh,
        compiler_params=pltpu.CompilerParams(needs_layout_passes=False),
        out_type=jax.ShapeDtypeStruct((_B * _PRE,), jnp.int32),
        scratch_types=[
            pltpu.VMEM((_PW,), jnp.float32),
            pltpu.VMEM((_PW,), jnp.int32),
            pltpu.VMEM((16,), jnp.int32),
            pltpu.VMEM((16, 16), jnp.int32),
            pltpu.VMEM((_PRE,), jnp.int32),
            pltpu.VMEM((128,), jnp.int32),
            pltpu.VMEM((128,), jnp.int32),
            pltpu.VMEM_SHARED((16, 16), jnp.int32),
            pltpu.VMEM_SHARED((16, _PRE), jnp.int32),
        ],
    )
    return f(scores_flat)


def _nms_kernel(tc_ref, tr_ref, rois_ref, rsc_ref, rlb_ref):
    gc = tc_ref[0]
    gr = tr_ref[0]
    dxc = jnp.abs(gc[:, 3:4])
    dyc = jnp.abs(gc[:, 4:5])
    x1c = gc[:, 0:1] - dxc * 0.5
    x2c = gc[:, 0:1] + dxc * 0.5
    y1c = gc[:, 1:2] - dyc * 0.5
    y2c = gc[:, 1:2] + dyc * 0.5
    ac = dxc * dyc
    dxr = jnp.abs(gr[3:4, :])
    dyr = jnp.abs(gr[4:5, :])
    x1r = gr[0:1, :] - dxr * 0.5
    x2r = gr[0:1, :] + dxr * 0.5
    y1r = gr[1:2, :] - dyr * 0.5
    y2r = gr[1:2, :] + dyr * 0.5
    ar = dxr * dyr
    ix = jnp.maximum(0.0, jnp.minimum(x2c, x2r) - jnp.maximum(x1c, x1r))
    iy = jnp.maximum(0.0, jnp.minimum(y2c, y2r) - jnp.maximum(y1c, y1r))
    inter = ix * iy
    iou = inter / jnp.maximum(ac + ar - inter, 1e-8)

    sc_col = gc[:, 7:8]
    sc_row = gr[7:8, :]
    ri = lax.broadcasted_iota(jnp.int32, (_PRE, _PRE), 0)
    ci = lax.broadcasted_iota(jnp.int32, (_PRE, _PRE), 1)
    dom = (sc_col > sc_row) | ((sc_col == sc_row) & (ri < ci))
    dom_f = jnp.where(dom, 1.0, 0.0)
    sup_m = jnp.where(dom & (iou > _THRESH), 1.0, 0.0)

    def cond(carry):
        return carry[1] == 1

    def body(carry):
        keep, _ = carry
        cnt = jnp.dot(keep, sup_m, preferred_element_type=jnp.float32)
        new = jnp.where(cnt > 0.0, 0.0, 1.0)
        changed = (jnp.sum(jnp.abs(new - keep)) > 0.0).astype(jnp.int32)
        return (new, changed)

    keep0 = jnp.ones((1, _PRE), jnp.float32)
    keep, _ = lax.while_loop(cond, body, (keep0, jnp.int32(1)))

    rank = jnp.dot(keep, dom_f, preferred_element_type=jnp.float32)
    slot = lax.broadcasted_iota(jnp.int32, (_POST, 1), 0).astype(jnp.float32)
    onehot = jnp.where((rank == slot) & (keep > 0.0), 1.0, 0.0)

    cols = [jnp.sum(onehot * gr[c:c + 1, :], axis=1, keepdims=True)
            for c in range(7)]
    rois_ref[0] = jnp.concatenate(cols, axis=1)
    rsc_ref[0] = jnp.sum(onehot * sc_row, axis=1, keepdims=True)
    lf = jnp.sum(onehot * gr[8:9, :], axis=1, keepdims=True)
    rlb_ref[0] = lf.astype(jnp.int32)


def _nms_from_member_table(table_col):
    B = table_col.shape[0]
    table_row = jnp.swapaxes(table_col, 1, 2)
    rois, rsc, rlb = pl.pallas_call(
        _nms_kernel,
        grid=(B,),
        in_specs=[
            pl.BlockSpec((1, _PRE, 9), lambda i: (i, 0, 0)),
            pl.BlockSpec((1, 9, _PRE), lambda i: (i, 0, 0)),
        ],
        out_specs=[
            pl.BlockSpec((1, _POST, 7), lambda i: (i, 0, 0)),
            pl.BlockSpec((1, _POST, 1), lambda i: (i, 0, 0)),
            pl.BlockSpec((1, _POST, 1), lambda i: (i, 0, 0)),
        ],
        out_shape=[
            jax.ShapeDtypeStruct((B, _POST, 7), jnp.float32),
            jax.ShapeDtypeStruct((B, _POST, 1), jnp.float32),
            jax.ShapeDtypeStruct((B, _POST, 1), jnp.int32),
        ],
    )(table_col, table_row)
    return rois, rsc[..., 0], rlb[..., 0]


def kernel(batch_box_preds, batch_cls_preds):
    B, N, _ = batch_box_preds.shape
    scores = jnp.max(batch_cls_preds, axis=-1)
    labels = jnp.argmax(batch_cls_preds, axis=-1)
    table = jnp.concatenate([
        batch_box_preds,
        scores[..., None],
        (labels + 1).astype(jnp.float32)[..., None],
    ], axis=-1)
    scores_pad = jnp.pad(scores, ((0, 0), (0, _NP - N)),
                         constant_values=-jnp.inf)
    member_idx = _sc_select(scores_pad.reshape(-1)).reshape(B, _PRE)
    member_table = jnp.take_along_axis(table, member_idx[..., None], axis=1)
    return _nms_from_member_table(member_table)
